# baseline (speedup 1.0000x reference)
import functools

import jax
import jax.numpy as jnp
from jax import lax
from jax.experimental import pallas as pl
from jax.experimental.pallas import tpu as pltpu
from jax.experimental.pallas import tpu_sc as plsc

DIM = 64
EPS = 1e-05
NC, NS = 2, 16
NW = NC * NS
CHUNK = 128
L = 16
NV = DIM // L


def _rsqrt(a):
    i = lax.bitcast_convert_type(a, jnp.int32)
    i = jnp.int32(0x5F3759DF) - lax.shift_right_arithmetic(i, 1)
    y = lax.bitcast_convert_type(i, jnp.float32)
    xh = a * 0.5
    y = y * (1.5 - xh * y * y)
    y = y * (1.5 - xh * y * y)
    y = y * (1.5 - xh * y * y)
    return y


def _make_call(nchunk):
    rows_per_w = nchunk * CHUNK
    total = NW * rows_per_w
    mesh = plsc.VectorSubcoreMesh(core_axis_name="c", subcore_axis_name="s")

    @functools.partial(
        pl.kernel,
        mesh=mesh,
        compiler_params=pltpu.CompilerParams(
            needs_layout_passes=False, use_tc_tiling_on_sc=False),
        out_type=jax.ShapeDtypeStruct((total, DIM), jnp.float32),
        scratch_types=[
            pltpu.VMEM((nchunk, CHUNK), jnp.int32),
            pltpu.VMEM((CHUNK, DIM), jnp.float32),
            pltpu.VMEM((2, DIM), jnp.float32),
            pltpu.SemaphoreType.DMA,
        ],
    )
    def call(idx_hbm, table_hbm, scale_hbm, bias_hbm, out_hbm,
             idx_v, rows_v, sb_v, sem):
        cid = lax.axis_index("c")
        sid = lax.axis_index("s")
        wid = sid * NC + cid

        pltpu.sync_copy(idx_hbm.at[wid], idx_v)
        pltpu.sync_copy(scale_hbm, sb_v.at[0])
        pltpu.sync_copy(bias_hbm, sb_v.at[1])
        sv = [sb_v[0, pl.ds(L * k, L)] for k in range(NV)]
        bv = [sb_v[1, pl.ds(L * k, L)] for k in range(NV)]
        out_base = wid * rows_per_w

        def chunk_body(c, carry):
            pltpu.async_copy(table_hbm.at[idx_v.at[c]], rows_v, sem).wait()

            def row_body(r, rcarry):
                xs = [rows_v[r, pl.ds(L * k, L)] for k in range(NV)]
                s = (xs[0] + xs[1]) + (xs[2] + xs[3])
                q = (xs[0] * xs[0] + xs[1] * xs[1]) + \
                    (xs[2] * xs[2] + xs[3] * xs[3])
                ts = jnp.sum(s)
                tq = jnp.sum(q)
                mean = ts * (1.0 / DIM)
                var = tq * (1.0 / DIM) - mean * mean
                rinv = _rsqrt(var + EPS)
                mr = mean * rinv
                rs = lax.broadcast_in_dim(rinv, (L,), ())
                mrv = lax.broadcast_in_dim(mr, (L,), ())
                for k in range(NV):
                    o = xs[k] * (sv[k] * rs) + (bv[k] - sv[k] * mrv)
                    rows_v[r, pl.ds(L * k, L)] = o
                return rcarry

            lax.fori_loop(0, CHUNK, row_body, 0)
            pltpu.sync_copy(
                rows_v, out_hbm.at[pl.ds(out_base + c * CHUNK, CHUNK)])
            return carry

        lax.fori_loop(0, nchunk, chunk_body, 0)

    return call


_CALLS = {}


def kernel(inputs, emb_weight, ln_scale, ln_bias):
    b, s = inputs.shape
    total = b * s
    assert total % (NW * CHUNK) == 0
    nchunk = total // (NW * CHUNK)
    if nchunk not in _CALLS:
        _CALLS[nchunk] = _make_call(nchunk)
    idx = inputs.astype(jnp.int32).reshape(NW, nchunk, CHUNK)
    out = _CALLS[nchunk](idx, emb_weight, ln_scale, ln_bias)
    return out.reshape(b, s, DIM)
